# Initial kernel scaffold; baseline (speedup 1.0000x reference)
import jax
import jax.numpy as jnp
from jax.experimental import pallas as pl


def kernel(input, embed):
    raise NotImplementedError("write your pallas kernel here")


# speedup vs baseline: 15.1133x; 15.1133x over previous
import jax
import jax.numpy as jnp
from jax.experimental import pallas as pl
from jax.experimental.pallas import tpu as pltpu
from jax.experimental.pallas import tpu_sc as plsc

EMB = 32
QBLK = 512


def _knn_body(q_ref, k_ref, ids_ref, loss_ref):
    i = pl.program_id(0)
    q = q_ref[...]
    k = k_ref[...]
    ksq = jnp.sum(k * k, axis=1)
    qsq = jnp.sum(q * q, axis=1)
    s = jax.lax.dot_general(k, q, (((1,), (1,)), ((), ())),
                            preferred_element_type=jnp.float32)
    s = s * 2.0
    s = s - ksq[:, None]
    s = s - qsq[None, :]
    ids = jnp.argmax(s, axis=0).astype(jnp.int32)
    maxv = jnp.max(s, axis=0)

    ids_ref[0, 0, :] = ids

    @pl.when(i == 0)
    def _():
        loss_ref[...] = jnp.zeros_like(loss_ref)

    loss_ref[...] += -jnp.sum(maxv)


def _knn_tc(flatten, embed):
    q, _ = flatten.shape
    k, _ = embed.shape
    g = q // QBLK
    ids3, loss = pl.pallas_call(
        _knn_body,
        grid=(g,),
        in_specs=[
            pl.BlockSpec((QBLK, EMB), lambda i: (i, 0)),
            pl.BlockSpec((k, EMB), lambda i: (0, 0)),
        ],
        out_specs=[
            pl.BlockSpec((1, 1, QBLK), lambda i: (i, 0, 0)),
            pl.BlockSpec((1, 1), lambda i: (0, 0)),
        ],
        out_shape=[
            jax.ShapeDtypeStruct((g, 1, QBLK), jnp.int32),
            jax.ShapeDtypeStruct((1, 1), jnp.float32),
        ],
    )(flatten, embed)
    return ids3.reshape(q), loss[0, 0]


def _sc_gather(embed_pad, ids_flat):
    n = ids_flat.shape[0]
    window = 128
    lanes = embed_pad.shape[1]
    mesh = plsc.VectorSubcoreMesh(core_axis_name="c", subcore_axis_name="s")
    ids2 = ids_flat.reshape(1, n)

    @pl.kernel(out_type=jax.ShapeDtypeStruct((n, lanes), embed_pad.dtype),
               mesh=mesh)
    def gather_kernel(x_hbm, i_hbm, o_hbm):
        def body(i_vmem, o_vmem):
            pltpu.sync_copy(x_hbm.at[i_vmem.at[0]], o_vmem)

        pltpu.emit_pipeline(
            body,
            grid=(n // window,),
            in_specs=[pl.BlockSpec((1, window), lambda i: (0, i))],
            out_specs=[pl.BlockSpec((window, lanes), lambda i: (i, 0))],
            core_axis_name=("c", "s"),
            dimension_semantics=(pltpu.PARALLEL,),
        )(i_hbm, o_hbm)

    return gather_kernel(embed_pad, ids2)


def kernel(input, embed):
    b, c, h, w = input.shape
    flatten = jnp.transpose(input, (0, 3, 2, 1)).reshape(-1, c)
    ids_flat, loss_sum = _knn_tc(flatten, embed)
    embed_pad = jnp.pad(embed, ((0, 0), (0, 128 - c)))
    quantized_flat = _sc_gather(embed_pad, ids_flat)[:, :c]
    commit_loss = loss_sum / input.size
    ids = ids_flat.reshape(b, h, w)
    quantized_ste = jnp.transpose(quantized_flat.reshape(b, w, h, c),
                                  (0, 3, 2, 1))
    return (quantized_ste, commit_loss, ids)


# speedup vs baseline: 16.3101x; 1.0792x over previous
import jax
import jax.numpy as jnp
from jax.experimental import pallas as pl
from jax.experimental.pallas import tpu as pltpu
from jax.experimental.pallas import tpu_sc as plsc

EMB = 32
QBLK = 1024


def _knn_body(q_ref, k_ref, ids_ref, loss_ref):
    i = pl.program_id(0)
    q = q_ref[...]
    k = k_ref[...]
    ksq = jnp.sum(k * k, axis=1)
    qsq = jnp.sum(q * q, axis=1)
    s = jax.lax.dot_general(k + k, q, (((1,), (1,)), ((), ())),
                            preferred_element_type=jnp.float32)
    s = s - ksq[:, None]
    s = s - qsq[None, :]
    ids = jnp.argmax(s, axis=0).astype(jnp.int32)
    maxv = jnp.max(s, axis=0)

    ids_ref[0, 0, :] = ids

    @pl.when(i == 0)
    def _():
        loss_ref[...] = jnp.zeros_like(loss_ref)

    loss_ref[...] += -jnp.sum(maxv)


def _knn_tc(flatten, embed):
    q, _ = flatten.shape
    k, _ = embed.shape
    g = q // QBLK
    ids3, loss = pl.pallas_call(
        _knn_body,
        grid=(g,),
        in_specs=[
            pl.BlockSpec((QBLK, EMB), lambda i: (i, 0)),
            pl.BlockSpec((k, EMB), lambda i: (0, 0)),
        ],
        out_specs=[
            pl.BlockSpec((1, 1, QBLK), lambda i: (i, 0, 0)),
            pl.BlockSpec((1, 1), lambda i: (0, 0)),
        ],
        out_shape=[
            jax.ShapeDtypeStruct((g, 1, QBLK), jnp.int32),
            jax.ShapeDtypeStruct((1, 1), jnp.float32),
        ],
    )(flatten, embed)
    return ids3.reshape(q), loss[0, 0]


def _sc_gather(embed_pad, ids_flat):
    n = ids_flat.shape[0]
    window = 128
    lanes = embed_pad.shape[1]
    mesh = plsc.VectorSubcoreMesh(core_axis_name="c", subcore_axis_name="s")
    ids2 = ids_flat.reshape(1, n)

    @pl.kernel(out_type=jax.ShapeDtypeStruct((n, lanes), embed_pad.dtype),
               mesh=mesh)
    def gather_kernel(x_hbm, i_hbm, o_hbm):
        def body(i_vmem, o_vmem):
            pltpu.sync_copy(x_hbm.at[i_vmem.at[0]], o_vmem)

        pltpu.emit_pipeline(
            body,
            grid=(n // window,),
            in_specs=[pl.BlockSpec((1, window), lambda i: (0, i))],
            out_specs=[pl.BlockSpec((window, lanes), lambda i: (i, 0))],
            core_axis_name=("c", "s"),
            dimension_semantics=(pltpu.PARALLEL,),
        )(i_hbm, o_hbm)

    return gather_kernel(embed_pad, ids2)


def kernel(input, embed):
    b, c, h, w = input.shape
    flatten = jnp.transpose(input, (0, 3, 2, 1)).reshape(-1, c)
    ids_flat, loss_sum = _knn_tc(flatten, embed)
    embed_pad = jnp.pad(embed, ((0, 0), (0, 128 - c)))
    quantized_flat = _sc_gather(embed_pad, ids_flat)[:, :c]
    commit_loss = loss_sum / input.size
    ids = ids_flat.reshape(b, h, w)
    quantized_ste = jnp.transpose(quantized_flat.reshape(b, w, h, c),
                                  (0, 3, 2, 1))
    return (quantized_ste, commit_loss, ids)


# speedup vs baseline: 16.4622x; 1.0093x over previous
import jax
import jax.numpy as jnp
from jax.experimental import pallas as pl
from jax.experimental.pallas import tpu as pltpu
from jax.experimental.pallas import tpu_sc as plsc

EMB = 32
QBLK = 1024
PADW = 128


def _knn_body(qc_ref, k_ref, ids_ref, loss_ref, ep_ref):
    i = pl.program_id(0)
    qc = qc_ref[0]
    k = k_ref[...]
    ksq = jnp.sum(k * k, axis=1)
    q = qc.T
    qsq = jnp.sum(q * q, axis=1)
    s = jax.lax.dot_general(k + k, qc, (((1,), (0,)), ((), ())),
                            preferred_element_type=jnp.float32)
    s = s - ksq[:, None]
    s = s - qsq[None, :]
    ids = jnp.argmax(s, axis=0).astype(jnp.int32)
    maxv = jnp.max(s, axis=0)

    ids_ref[0, 0, :] = ids

    @pl.when(i == 0)
    def _():
        loss_ref[...] = jnp.zeros_like(loss_ref)
        ep_ref[:, :EMB] = k
        ep_ref[:, EMB:] = jnp.zeros_like(ep_ref[:, EMB:])

    loss_ref[...] += -jnp.sum(maxv)


def _knn_tc(qc_all, embed):
    b, _, hw = qc_all.shape
    nq = b * hw
    kk, _ = embed.shape
    g = nq // QBLK
    ids3, loss, embed_pad = pl.pallas_call(
        _knn_body,
        grid=(g,),
        in_specs=[
            pl.BlockSpec((1, EMB, QBLK), lambda i: (i, 0, 0)),
            pl.BlockSpec((kk, EMB), lambda i: (0, 0)),
        ],
        out_specs=[
            pl.BlockSpec((1, 1, QBLK), lambda i: (i, 0, 0)),
            pl.BlockSpec((1, 1), lambda i: (0, 0)),
            pl.BlockSpec((kk, PADW), lambda i: (0, 0)),
        ],
        out_shape=[
            jax.ShapeDtypeStruct((g, 1, QBLK), jnp.int32),
            jax.ShapeDtypeStruct((1, 1), jnp.float32),
            jax.ShapeDtypeStruct((kk, PADW), jnp.float32),
        ],
    )(qc_all, embed)
    return ids3.reshape(nq), loss[0, 0], embed_pad


def _sc_gather(embed_pad, ids_flat):
    n = ids_flat.shape[0]
    window = 128
    lanes = embed_pad.shape[1]
    mesh = plsc.VectorSubcoreMesh(core_axis_name="c", subcore_axis_name="s")
    ids2 = ids_flat.reshape(1, n)

    @pl.kernel(out_type=jax.ShapeDtypeStruct((n, lanes), embed_pad.dtype),
               mesh=mesh)
    def gather_kernel(x_hbm, i_hbm, o_hbm):
        def body(i_vmem, o_vmem):
            pltpu.sync_copy(x_hbm.at[i_vmem.at[0]], o_vmem)

        pltpu.emit_pipeline(
            body,
            grid=(n // window,),
            in_specs=[pl.BlockSpec((1, window), lambda i: (0, i))],
            out_specs=[pl.BlockSpec((window, lanes), lambda i: (i, 0))],
            core_axis_name=("c", "s"),
            dimension_semantics=(pltpu.PARALLEL,),
        )(i_hbm, o_hbm)

    return gather_kernel(embed_pad, ids2)


def kernel(input, embed):
    b, c, h, w = input.shape
    qc_all = input.reshape(b, c, h * w)
    ids_hw, loss_sum, embed_pad = _knn_tc(qc_all, embed)
    gathered = _sc_gather(embed_pad, ids_hw)
    commit_loss = loss_sum / input.size
    ids = jnp.transpose(ids_hw.reshape(b, h, w), (0, 2, 1))
    quantized_ste = jnp.transpose(gathered.reshape(b, h, w, PADW)[..., :c],
                                  (0, 3, 1, 2))
    return (quantized_ste, commit_loss, ids)


# speedup vs baseline: 16.6437x; 1.0110x over previous
import jax
import jax.numpy as jnp
from jax.experimental import pallas as pl
from jax.experimental.pallas import tpu as pltpu
from jax.experimental.pallas import tpu_sc as plsc

EMB = 32
QBLK = 1024
PADW = 128


def _knn_body(qc_ref, k_ref, ids_ref, loss_ref, ep_ref):
    i = pl.program_id(0)
    qc = qc_ref[0]
    k = k_ref[...]
    ksq = jnp.sum(k * k, axis=1)
    q = qc.T
    qsq = jnp.sum(q * q, axis=1)
    s = jax.lax.dot_general(k + k, qc, (((1,), (0,)), ((), ())),
                            preferred_element_type=jnp.float32)
    s = s - ksq[:, None]
    s = s - qsq[None, :]
    ids = jnp.argmax(s, axis=0).astype(jnp.int32)
    maxv = jnp.max(s, axis=0)

    ids_ref[0, 0, :] = ids

    @pl.when(i == 0)
    def _():
        loss_ref[...] = jnp.zeros_like(loss_ref)
        ep_ref[:, :EMB] = k

    loss_ref[...] += -jnp.sum(maxv)


def _knn_tc(qc_all, embed):
    b, _, hw = qc_all.shape
    nq = b * hw
    kk, _ = embed.shape
    g = nq // QBLK
    ids3, loss, embed_pad = pl.pallas_call(
        _knn_body,
        grid=(g,),
        in_specs=[
            pl.BlockSpec((1, EMB, QBLK), lambda i: (i, 0, 0)),
            pl.BlockSpec((kk, EMB), lambda i: (0, 0)),
        ],
        out_specs=[
            pl.BlockSpec((1, 1, QBLK), lambda i: (i, 0, 0)),
            pl.BlockSpec((1, 1), lambda i: (0, 0)),
            pl.BlockSpec((kk, PADW), lambda i: (0, 0)),
        ],
        out_shape=[
            jax.ShapeDtypeStruct((g, 1, QBLK), jnp.int32),
            jax.ShapeDtypeStruct((1, 1), jnp.float32),
            jax.ShapeDtypeStruct((kk, PADW), jnp.float32),
        ],
    )(qc_all, embed)
    return ids3.reshape(nq), loss[0, 0], embed_pad


def _sc_gather(embed_pad, ids_flat):
    n = ids_flat.shape[0]
    window = 256
    lanes = embed_pad.shape[1]
    mesh = plsc.VectorSubcoreMesh(core_axis_name="c", subcore_axis_name="s")
    ids2 = ids_flat.reshape(1, n)

    @pl.kernel(out_type=jax.ShapeDtypeStruct((n, lanes), embed_pad.dtype),
               mesh=mesh)
    def gather_kernel(x_hbm, i_hbm, o_hbm):
        def body(i_vmem, o_vmem):
            pltpu.sync_copy(x_hbm.at[i_vmem.at[0]], o_vmem)

        pltpu.emit_pipeline(
            body,
            grid=(n // window,),
            in_specs=[pl.BlockSpec((1, window), lambda i: (0, i))],
            out_specs=[pl.BlockSpec((window, lanes), lambda i: (i, 0))],
            core_axis_name=("c", "s"),
            dimension_semantics=(pltpu.PARALLEL,),
        )(i_hbm, o_hbm)

    return gather_kernel(embed_pad, ids2)


def kernel(input, embed):
    b, c, h, w = input.shape
    qc_all = input.reshape(b, c, h * w)
    ids_hw, loss_sum, embed_pad = _knn_tc(qc_all, embed)
    gathered = _sc_gather(embed_pad, ids_hw)
    commit_loss = loss_sum / input.size
    ids = jnp.transpose(ids_hw.reshape(b, h, w), (0, 2, 1))
    quantized_ste = jnp.transpose(gathered.reshape(b, h, w, PADW)[..., :c],
                                  (0, 3, 1, 2))
    return (quantized_ste, commit_loss, ids)
